# baseline (speedup 1.0000x reference)
import functools

import jax
import jax.numpy as jnp
from jax import lax
from jax.experimental import pallas as pl
from jax.experimental.pallas import tpu as pltpu
from jax.experimental.pallas import tpu_sc as plsc

_ROWS = 512
_COLS = 16384
_R_TC = 512
_R_SC = _ROWS - _R_TC
_TC_BLOCK = 8

_NC = 2
_NS = 16
_NW = _NC * _NS
_RPW = _R_SC // _NW
_L = 16
_STEPS = _COLS // _L
_U = 8


def _tc_body(x_ref, t_ref, o_ref):
    i = pl.program_id(0)
    x = 0.5 * (1.0 + jnp.tanh(0.5 * x_ref[...]))
    t = t_ref[...]
    p = jnp.sum(x * t, axis=1)
    s = jnp.sum(x, axis=1)
    ts = jnp.sum(t, axis=1)
    loss = 1.0 - (2.0 * p + 1.0) / (s + ts + 1.0)
    blk = jnp.sum(loss).reshape(1, 1)

    @pl.when(i == 0)
    def _():
        o_ref[...] = jnp.zeros((1, 1), jnp.float32)

    o_ref[...] += blk


def _sc_body(x_hbm, t_hbm, out_hbm, xbuf, tbuf, lbuf, sx0, sx1, st0, st1):
    wid = lax.axis_index("s") * _NC + lax.axis_index("c")
    base = _R_TC + wid * _RPW
    sx = [sx0, sx1]
    st = [st0, st1]
    pltpu.async_copy(x_hbm.at[base], xbuf.at[0], sx[0])
    pltpu.async_copy(t_hbm.at[base], tbuf.at[0], st[0])
    pltpu.async_copy(x_hbm.at[base + 1], xbuf.at[1], sx[1])
    pltpu.async_copy(t_hbm.at[base + 1], tbuf.at[1], st[1])
    lane = lax.iota(jnp.int32, _L)
    last = base + _RPW - 1

    def pair_body(i, loss_vec):
        for s in (0, 1):
            r = 2 * i + s
            pltpu.make_async_copy(x_hbm.at[base], xbuf.at[s], sx[s]).wait()
            pltpu.make_async_copy(t_hbm.at[base], tbuf.at[s], st[s]).wait()

            def body(j, carry, s=s):
                acc_s, acc_p, acc_t = carry
                base_off = j * (_L * _U)
                sigs, sigts, ts = [], [], []
                for k in range(_U):
                    x = xbuf[s, pl.ds(base_off + k * _L, _L)]
                    t = tbuf[s, pl.ds(base_off + k * _L, _L)]
                    sig = 1.0 / (1.0 + jnp.exp(-x))
                    sigs.append(sig)
                    sigts.append(sig * t)
                    ts.append(t)

                def tree(vs):
                    while len(vs) > 1:
                        vs = [a + b for a, b in zip(vs[0::2], vs[1::2])]
                    return vs[0]

                return (acc_s + tree(sigs), acc_p + tree(sigts), acc_t + tree(ts))

            z = jnp.zeros((_L,), jnp.float32)
            acc_s, acc_p, acc_t = lax.fori_loop(0, _STEPS // _U, body, (z, z, z))
            nxt = jnp.minimum(base + r + 2, last)
            pltpu.async_copy(x_hbm.at[nxt], xbuf.at[s], sx[s])
            pltpu.async_copy(t_hbm.at[nxt], tbuf.at[s], st[s])
            sv = jnp.full((_L,), jnp.sum(acc_s))
            pv = jnp.full((_L,), jnp.sum(acc_p))
            tv = jnp.full((_L,), jnp.sum(acc_t))
            loss_v = 1.0 - (2.0 * pv + 1.0) / (sv + tv + 1.0)
            loss_vec = jnp.where(lane == r, loss_v, loss_vec)
        return loss_vec

    loss_vec = lax.fori_loop(0, _RPW // 2, pair_body, jnp.zeros((_L,), jnp.float32))
    for s in (0, 1):
        pltpu.make_async_copy(x_hbm.at[base], xbuf.at[s], sx[s]).wait()
        pltpu.make_async_copy(t_hbm.at[base], tbuf.at[s], st[s]).wait()
    lbuf[...] = loss_vec
    pltpu.sync_copy(lbuf, out_hbm.at[wid])


_sc_call = None if _R_SC == 0 else pl.kernel(
    _sc_body,
    out_type=jax.ShapeDtypeStruct((_NW, _L), jnp.float32),
    mesh=plsc.VectorSubcoreMesh(core_axis_name="c", subcore_axis_name="s"),
    compiler_params=pltpu.CompilerParams(
        needs_layout_passes=False, skip_device_barrier=True),
    scratch_types=[
        pltpu.VMEM((2, _COLS), jnp.float32),
        pltpu.VMEM((2, _COLS), jnp.float32),
        pltpu.VMEM((_L,), jnp.float32),
        pltpu.SemaphoreType.DMA,
        pltpu.SemaphoreType.DMA,
        pltpu.SemaphoreType.DMA,
        pltpu.SemaphoreType.DMA,
    ],
)


def kernel(inputs, targets, num_boxes):
    sc_out = _sc_call(inputs, targets) if _R_SC else jnp.zeros((1,), jnp.float32)
    tc_out = pl.pallas_call(
        _tc_body,
        grid=(_R_TC // _TC_BLOCK,),
        in_specs=[
            pl.BlockSpec((_TC_BLOCK, _COLS), lambda i: (i, 0)),
            pl.BlockSpec((_TC_BLOCK, _COLS), lambda i: (i, 0)),
        ],
        out_specs=pl.BlockSpec((1, 1), lambda i: (0, 0)),
        out_shape=jax.ShapeDtypeStruct((1, 1), jnp.float32),
    )(inputs, targets)
    total = jnp.sum(sc_out) + tc_out[0, 0]
    return total / (num_boxes + 1e-06)


# speedup vs baseline: 1.9643x; 1.9643x over previous
import functools

import jax
import jax.numpy as jnp
from jax import lax
from jax.experimental import pallas as pl
from jax.experimental.pallas import tpu as pltpu
from jax.experimental.pallas import tpu_sc as plsc

_ROWS = 512
_COLS = 16384
_R_TC = 512
_R_SC = _ROWS - _R_TC
_TC_BLOCK = 32

_NC = 2
_NS = 16
_NW = _NC * _NS
_RPW = _R_SC // _NW
_L = 16
_STEPS = _COLS // _L
_U = 8


def _tc_body(x_ref, t_ref, o_ref):
    i = pl.program_id(0)
    x = 0.5 * (1.0 + jnp.tanh(0.5 * x_ref[...]))
    t = t_ref[...]
    p = jnp.sum(x * t, axis=1)
    s = jnp.sum(x, axis=1)
    ts = jnp.sum(t, axis=1)
    loss = 1.0 - (2.0 * p + 1.0) / (s + ts + 1.0)
    blk = jnp.sum(loss).reshape(1, 1)

    @pl.when(i == 0)
    def _():
        o_ref[...] = jnp.zeros((1, 1), jnp.float32)

    o_ref[...] += blk


def _sc_body(x_hbm, t_hbm, out_hbm, xbuf, tbuf, lbuf, sx0, sx1, st0, st1):
    wid = lax.axis_index("s") * _NC + lax.axis_index("c")
    base = _R_TC + wid * _RPW
    sx = [sx0, sx1]
    st = [st0, st1]
    pltpu.async_copy(x_hbm.at[base], xbuf.at[0], sx[0])
    pltpu.async_copy(t_hbm.at[base], tbuf.at[0], st[0])
    pltpu.async_copy(x_hbm.at[base + 1], xbuf.at[1], sx[1])
    pltpu.async_copy(t_hbm.at[base + 1], tbuf.at[1], st[1])
    lane = lax.iota(jnp.int32, _L)
    last = base + _RPW - 1

    def pair_body(i, loss_vec):
        for s in (0, 1):
            r = 2 * i + s
            pltpu.make_async_copy(x_hbm.at[base], xbuf.at[s], sx[s]).wait()
            pltpu.make_async_copy(t_hbm.at[base], tbuf.at[s], st[s]).wait()

            def body(j, carry, s=s):
                acc_s, acc_p, acc_t = carry
                base_off = j * (_L * _U)
                sigs, sigts, ts = [], [], []
                for k in range(_U):
                    x = xbuf[s, pl.ds(base_off + k * _L, _L)]
                    t = tbuf[s, pl.ds(base_off + k * _L, _L)]
                    sig = 1.0 / (1.0 + jnp.exp(-x))
                    sigs.append(sig)
                    sigts.append(sig * t)
                    ts.append(t)

                def tree(vs):
                    while len(vs) > 1:
                        vs = [a + b for a, b in zip(vs[0::2], vs[1::2])]
                    return vs[0]

                return (acc_s + tree(sigs), acc_p + tree(sigts), acc_t + tree(ts))

            z = jnp.zeros((_L,), jnp.float32)
            acc_s, acc_p, acc_t = lax.fori_loop(0, _STEPS // _U, body, (z, z, z))
            nxt = jnp.minimum(base + r + 2, last)
            pltpu.async_copy(x_hbm.at[nxt], xbuf.at[s], sx[s])
            pltpu.async_copy(t_hbm.at[nxt], tbuf.at[s], st[s])
            sv = jnp.full((_L,), jnp.sum(acc_s))
            pv = jnp.full((_L,), jnp.sum(acc_p))
            tv = jnp.full((_L,), jnp.sum(acc_t))
            loss_v = 1.0 - (2.0 * pv + 1.0) / (sv + tv + 1.0)
            loss_vec = jnp.where(lane == r, loss_v, loss_vec)
        return loss_vec

    loss_vec = lax.fori_loop(0, _RPW // 2, pair_body, jnp.zeros((_L,), jnp.float32))
    for s in (0, 1):
        pltpu.make_async_copy(x_hbm.at[base], xbuf.at[s], sx[s]).wait()
        pltpu.make_async_copy(t_hbm.at[base], tbuf.at[s], st[s]).wait()
    lbuf[...] = loss_vec
    pltpu.sync_copy(lbuf, out_hbm.at[wid])


_sc_call = None if _R_SC == 0 else pl.kernel(
    _sc_body,
    out_type=jax.ShapeDtypeStruct((_NW, _L), jnp.float32),
    mesh=plsc.VectorSubcoreMesh(core_axis_name="c", subcore_axis_name="s"),
    compiler_params=pltpu.CompilerParams(
        needs_layout_passes=False, skip_device_barrier=True),
    scratch_types=[
        pltpu.VMEM((2, _COLS), jnp.float32),
        pltpu.VMEM((2, _COLS), jnp.float32),
        pltpu.VMEM((_L,), jnp.float32),
        pltpu.SemaphoreType.DMA,
        pltpu.SemaphoreType.DMA,
        pltpu.SemaphoreType.DMA,
        pltpu.SemaphoreType.DMA,
    ],
)


def kernel(inputs, targets, num_boxes):
    sc_out = _sc_call(inputs, targets) if _R_SC else jnp.zeros((1,), jnp.float32)
    tc_out = pl.pallas_call(
        _tc_body,
        grid=(_R_TC // _TC_BLOCK,),
        in_specs=[
            pl.BlockSpec((_TC_BLOCK, _COLS), lambda i: (i, 0)),
            pl.BlockSpec((_TC_BLOCK, _COLS), lambda i: (i, 0)),
        ],
        out_specs=pl.BlockSpec((1, 1), lambda i: (0, 0)),
        out_shape=jax.ShapeDtypeStruct((1, 1), jnp.float32),
    )(inputs, targets)
    total = jnp.sum(sc_out) + tc_out[0, 0]
    return total / (num_boxes + 1e-06)


# speedup vs baseline: 2.2680x; 1.1546x over previous
import functools

import jax
import jax.numpy as jnp
from jax import lax
from jax.experimental import pallas as pl
from jax.experimental.pallas import tpu as pltpu

_ROWS = 512
_COLS = 16384
_CHUNKS = [8, 8, 16, 32] + [64] * 7
_NBUF = 4
_BUFROWS = 64


def _row_starts():
    starts, r = [], 0
    for c in _CHUNKS:
        starts.append(r)
        r += c
    assert r == _ROWS
    return starts


_STARTS = _row_starts()


def _tc_body(x_hbm, t_hbm, o_ref, xbuf, tbuf, *sems):
    xsem = sems[:_NBUF]
    tsem = sems[_NBUF:]

    def issue(ci):
        s = ci % _NBUF
        r0, nr = _STARTS[ci], _CHUNKS[ci]
        pltpu.async_copy(x_hbm.at[pl.ds(r0, nr), :], xbuf.at[s, pl.ds(0, nr)], xsem[s])
        pltpu.async_copy(t_hbm.at[pl.ds(r0, nr), :], tbuf.at[s, pl.ds(0, nr)], tsem[s])

    for ci in range(_NBUF):
        issue(ci)

    total = jnp.zeros((1, 1), jnp.float32)
    for ci in range(len(_CHUNKS)):
        s = ci % _NBUF
        nr = _CHUNKS[ci]
        pltpu.make_async_copy(
            x_hbm.at[pl.ds(0, nr), :], xbuf.at[s, pl.ds(0, nr)], xsem[s]).wait()
        pltpu.make_async_copy(
            t_hbm.at[pl.ds(0, nr), :], tbuf.at[s, pl.ds(0, nr)], tsem[s]).wait()
        x = 0.5 * (1.0 + jnp.tanh(0.5 * xbuf[s, pl.ds(0, nr)]))
        t = tbuf[s, pl.ds(0, nr)]
        p = jnp.sum(x * t, axis=1)
        sm = jnp.sum(x, axis=1)
        ts = jnp.sum(t, axis=1)
        loss = 1.0 - (2.0 * p + 1.0) / (sm + ts + 1.0)
        total = total + jnp.sum(loss).reshape(1, 1)
        if ci + _NBUF < len(_CHUNKS):
            issue(ci + _NBUF)
    o_ref[...] = total


def kernel(inputs, targets, num_boxes):
    tc_out = pl.pallas_call(
        _tc_body,
        in_specs=[
            pl.BlockSpec(memory_space=pl.ANY),
            pl.BlockSpec(memory_space=pl.ANY),
        ],
        out_specs=pl.BlockSpec(memory_space=pltpu.VMEM),
        out_shape=jax.ShapeDtypeStruct((1, 1), jnp.float32),
        scratch_shapes=(
            [pltpu.VMEM((_NBUF, _BUFROWS, _COLS), jnp.float32)] * 2
            + [pltpu.SemaphoreType.DMA] * (2 * _NBUF)
        ),
    )(inputs, targets)
    return tc_out[0, 0] / (num_boxes + 1e-06)
